# baseline (speedup 1.0000x reference)
import functools

import jax
import jax.numpy as jnp
from jax.experimental import pallas as pl
from jax.experimental.pallas import tpu as pltpu


def _compute_slab(vb, w, *, S, d):
    cols = jax.lax.broadcasted_iota(jnp.int32, (d, S), 1)
    values = jnp.max(vb, axis=1, keepdims=True)
    idx = jnp.min(jnp.where(vb == values, cols, S), axis=1,
                  keepdims=True)
    v_cls = vb[:, 0:1]
    out = jnp.where(cols == idx, v_cls, vb)
    out = out * w
    col0 = jnp.where(cols == 0, values * w[0:1, 0:1], out)
    return col0


def _swd7_body(m_ref, v_hbm, o_hbm, ibuf, obuf, isem, osem, *, N, S, d):
    i = pl.program_id(0)
    w = 1.0 - m_ref[0]

    @pl.when(i == 0)
    def _prologue():
        pltpu.make_async_copy(v_hbm.at[0], ibuf.at[0], isem.at[0]).start()

    @pl.when(i + 1 < N)
    def _prefetch():
        pltpu.make_async_copy(
            v_hbm.at[i + 1], ibuf.at[(i + 1) % 2], isem.at[(i + 1) % 2]
        ).start()

    pltpu.make_async_copy(v_hbm.at[i], ibuf.at[i % 2], isem.at[i % 2]).wait()

    @pl.when(i >= 2)
    def _drain_old_store():
        pltpu.make_async_copy(obuf.at[i % 2], o_hbm.at[i - 2],
                              osem.at[i % 2]).wait()

    obuf[i % 2] = _compute_slab(ibuf[i % 2], w, S=S, d=d)
    pltpu.make_async_copy(obuf.at[i % 2], o_hbm.at[i], osem.at[i % 2]).start(
        priority=1)

    @pl.when(i == N - 1)
    def _epilogue():
        pltpu.make_async_copy(obuf.at[(i - 1) % 2], o_hbm.at[i - 1],
                              osem.at[(i - 1) % 2]).wait()
        pltpu.make_async_copy(obuf.at[i % 2], o_hbm.at[i],
                              osem.at[i % 2]).wait()


def kernel(q, k, v, attn_mask):
    del q, k
    B, H, S, d = v.shape
    N = B * H
    vt = jnp.swapaxes(v, 2, 3).reshape(N, d, S)
    mf = attn_mask.astype(jnp.float32).reshape(N, 1, S)
    out = pl.pallas_call(
        functools.partial(_swd7_body, N=N, S=S, d=d),
        grid=(N,),
        in_specs=[
            pl.BlockSpec((1, 1, S), lambda i: (i, 0, 0)),
            pl.BlockSpec(memory_space=pl.ANY),
        ],
        out_specs=pl.BlockSpec(memory_space=pl.ANY),
        out_shape=jax.ShapeDtypeStruct((N, d, S), v.dtype),
        scratch_shapes=[
            pltpu.VMEM((2, d, S), v.dtype),
            pltpu.VMEM((2, d, S), v.dtype),
            pltpu.SemaphoreType.DMA((2,)),
            pltpu.SemaphoreType.DMA((2,)),
        ],
    )(mf, vt)
    return jnp.swapaxes(out.reshape(B, H, d, S), 2, 3)
